# baseline (speedup 1.0000x reference)
import functools

import jax
import jax.numpy as jnp
from jax import lax
from jax.experimental import pallas as pl
from jax.experimental.pallas import tpu as pltpu
from jax.experimental.pallas import tpu_sc as plsc

_D = 32
_B = 16384
_K = 10
_NC = 2
_NS = 16
_NW = _NC * _NS
_BPW = _B // _NW
_CB = 64
_NCHUNK = _BPW // _CB
_G = _CB // 16

_sc_mesh = plsc.VectorSubcoreMesh(core_axis_name="c", subcore_axis_name="s")


@functools.partial(
    pl.kernel,
    out_type=(
        jax.ShapeDtypeStruct((_B,), jnp.float32),
        jax.ShapeDtypeStruct((_B * _K,), jnp.float32),
    ),
    mesh=_sc_mesh,
    scratch_types=(
        pltpu.VMEM((_CB,), jnp.int32),
        pltpu.VMEM((_CB,), jnp.int32),
        pltpu.VMEM((_CB,), jnp.int32),
        pltpu.VMEM((_CB,), jnp.int32),
        pltpu.VMEM((_CB * _K,), jnp.int32),
        pltpu.VMEM((_CB * _K,), jnp.int32),
        pltpu.VMEM((_CB, 4 * _D), jnp.float32),
        pltpu.VMEM((_CB, 4 * _D), jnp.float32),
        pltpu.VMEM((_CB * _K, 4 * _D), jnp.float32),
        pltpu.VMEM((_BPW,), jnp.float32),
        pltpu.VMEM((_BPW * _K,), jnp.float32),
        pltpu.SemaphoreType.DMA,
    ),
    compiler_params=pltpu.CompilerParams(needs_layout_passes=False),
)
def _sc_scores(tq, tcb, cq, ccb, nq, ncb, ttab, ctab, pos_out, neg_out,
               tq_v, tcb_v, cq_v, ccb_v, nq_v, ncb_v,
               trow_v, crow_v, nrow_v, pos_v, neg_v, sem):
    wid = lax.axis_index("s") * _NC + lax.axis_index("c")
    base = wid * _BPW
    ji = lax.iota(jnp.int32, 16)

    def chunk_body(c, carry):
        boff = base + c * _CB
        pltpu.sync_copy(tq.at[pl.ds(boff, _CB)], tq_v)
        pltpu.sync_copy(tcb.at[pl.ds(boff, _CB)], tcb_v)
        pltpu.sync_copy(cq.at[pl.ds(boff, _CB)], cq_v)
        pltpu.sync_copy(ccb.at[pl.ds(boff, _CB)], ccb_v)
        pltpu.sync_copy(nq.at[pl.ds(boff * _K, _CB * _K)], nq_v)
        pltpu.sync_copy(ncb.at[pl.ds(boff * _K, _CB * _K)], ncb_v)
        pltpu.async_copy(ttab.at[tq_v], trow_v, sem).wait()
        pltpu.async_copy(ctab.at[cq_v], crow_v, sem).wait()
        pltpu.async_copy(ctab.at[nq_v], nrow_v, sem).wait()
        for g in range(_G):
            rows = ji + g * 16
            tcbv = tcb_v[pl.ds(g * 16, 16)]
            ccbv = ccb_v[pl.ds(g * 16, 16)]
            nrows = []
            ncbvs = []
            for k in range(_K):
                nr = ji * _K + (k + g * 16 * _K)
                nrows.append(nr)
                ncbvs.append(plsc.load_gather(ncb_v, [nr]))

            def dot_body(d, accs):
                ap = accs[0]
                an = accs[1]
                td = plsc.load_gather(trow_v, [rows, tcbv + d])
                cd = plsc.load_gather(crow_v, [rows, ccbv + d])
                ap = ap + td * cd
                an_new = tuple(
                    an[k] + td * plsc.load_gather(nrow_v, [nrows[k], ncbvs[k] + d])
                    for k in range(_K))
                return (ap, an_new)

            zero = jnp.zeros((16,), jnp.float32)
            acc_pos, acc_neg = lax.fori_loop(
                0, _D, dot_body, (zero, (zero,) * _K))
            loc = c * _CB + g * 16
            pos_v[pl.ds(loc, 16)] = acc_pos
            for k in range(_K):
                plsc.store_scatter(neg_v, [(loc + ji) * _K + k], acc_neg[k])
        return carry

    lax.fori_loop(0, _NCHUNK, chunk_body, 0)
    pltpu.sync_copy(pos_v, pos_out.at[pl.ds(base, _BPW)])
    pltpu.sync_copy(neg_v, neg_out.at[pl.ds(base * _K, _BPW * _K)])


def _tc_loss_body(p_ref, n_ref, o_ref):
    s = jnp.sum(jax.nn.log_sigmoid(p_ref[...]))
    s = s + jnp.sum(jax.nn.log_sigmoid(-n_ref[...]))
    o_ref[...] = jnp.zeros_like(o_ref) - s


_tc_loss = pl.pallas_call(
    _tc_loss_body,
    out_shape=jax.ShapeDtypeStruct((1, 1), jnp.float32),
)


def kernel(target_idx, context_idx, neg_idx, emb_target_table, emb_context_table):
    tix = target_idx.astype(jnp.int32)
    cix = context_idx.astype(jnp.int32)
    nix = neg_idx.astype(jnp.int32).reshape(-1)
    pos, neg = _sc_scores(
        tix >> 2, (tix & 3) * _D,
        cix >> 2, (cix & 3) * _D,
        nix >> 2, (nix & 3) * _D,
        emb_target_table.reshape(-1, 4 * _D),
        emb_context_table.reshape(-1, 4 * _D),
    )
    out = _tc_loss(pos.reshape(128, 128), neg.reshape(1280, 128))
    return out.reshape(())
